# baseline (speedup 1.0000x reference)
import functools

import jax
import jax.numpy as jnp
from jax import lax
from jax.experimental import pallas as pl
from jax.experimental.pallas import tpu as pltpu
from jax.experimental.pallas import tpu_sc as plsc

B = 4096
S = 200
E = 64
C = 2
V = 1000000

VB = 65536
NW = 32
BPW = B // NW
NJ = BPW // 16


def _proj_body(w_ref, tt_ref, p_ref):
    w = w_ref[...] * (1.0 / S)
    chunk = tt_ref[...]
    res = lax.dot_general(w, chunk, (((0,), (0,)), ((), ())),
                          preferred_element_type=jnp.float32)
    u0 = lax.bitcast_convert_type(res[0].astype(jnp.bfloat16),
                                  jnp.uint16).astype(jnp.uint32)
    u1 = lax.bitcast_convert_type(res[1].astype(jnp.bfloat16),
                                  jnp.uint16).astype(jnp.uint32)
    p_ref[...] = (u0 | (u1 << 16)).astype(jnp.int32)


def _sc_pool(ids_hbm, mask_hbm, p_hbm, bias_hbm, out_hbm,
             idx_v, mask_v, r_v, bias_v, out_v, gsem):
    wid = lax.axis_index("s") * 2 + lax.axis_index("c")
    base = wid * BPW

    pltpu.sync_copy(ids_hbm.at[:, pl.ds(base, BPW)], idx_v)
    pltpu.sync_copy(mask_hbm.at[:, pl.ds(base, BPW)], mask_v)
    pltpu.sync_copy(bias_hbm, bias_v)

    def issue_body(s, _):
        pltpu.async_copy(p_hbm.at[idx_v.at[s, :]], r_v.at[s, :], gsem)
        return 0

    lax.fori_loop(0, S, issue_body, 0)

    zero = jnp.zeros((16,), jnp.float32)

    def drain_body(s, accs):
        pltpu.make_async_copy(p_hbm.at[idx_v.at[s, :]], r_v.at[s, :],
                              gsem).wait()
        out = []
        for j in range(NJ):
            m = mask_v[s, pl.ds(j * 16, 16)]
            pair = plsc.bitcast(r_v[s, pl.ds(j * 16, 16)], jnp.bfloat16)
            p0h, p1h = plsc.unpack(pair, format=plsc.PackFormat.INTERLEAVED)
            out.append(accs[2 * j] + m * p0h.astype(jnp.float32))
            out.append(accs[2 * j + 1] + m * p1h.astype(jnp.float32))
        return tuple(out)

    accs = lax.fori_loop(0, S, drain_body, (zero,) * (2 * NJ))
    b0 = bias_v[0, pl.ds(0, 16)]
    b1 = bias_v[1, pl.ds(0, 16)]
    for j in range(NJ):
        out_v[0, pl.ds(j * 16, 16)] = accs[2 * j] + b0
        out_v[1, pl.ds(j * 16, 16)] = accs[2 * j + 1] + b1
    pltpu.sync_copy(out_v, out_hbm.at[:, pl.ds(base, BPW)])


def kernel(input_ids, mask, emb_table, fc_w, fc_b):
    ids_t = input_ids.astype(jnp.int32).T
    mask_t = mask.T

    nblocks = pl.cdiv(V, VB)
    p_packed = pl.pallas_call(
        _proj_body,
        grid=(nblocks,),
        in_specs=[
            pl.BlockSpec((E, C), lambda i: (0, 0)),
            pl.BlockSpec((E, VB), lambda i: (0, i)),
        ],
        out_specs=pl.BlockSpec((VB,), lambda i: (i,)),
        out_shape=jax.ShapeDtypeStruct((V,), jnp.int32),
    )(fc_w, emb_table.T)

    mesh = plsc.VectorSubcoreMesh(core_axis_name="c", subcore_axis_name="s")
    pooled = pl.kernel(
        _sc_pool,
        out_type=jax.ShapeDtypeStruct((C, B), jnp.float32),
        mesh=mesh,
        compiler_params=pltpu.CompilerParams(needs_layout_passes=False,
                                             use_tc_tiling_on_sc=False),
        scratch_types=[
            pltpu.VMEM((S, BPW), jnp.int32),
            pltpu.VMEM((S, BPW), jnp.float32),
            pltpu.VMEM((S, BPW), jnp.int32),
            pltpu.VMEM((C, 16), jnp.float32),
            pltpu.VMEM((C, BPW), jnp.float32),
            pltpu.SemaphoreType.DMA,
        ],
    )(ids_t, mask_t, p_packed, jnp.broadcast_to(fc_b.reshape(C, 1), (C, 16)))

    return pooled.T


# speedup vs baseline: 1.0188x; 1.0188x over previous
import functools

import jax
import jax.numpy as jnp
from jax import lax
from jax.experimental import pallas as pl
from jax.experimental.pallas import tpu as pltpu
from jax.experimental.pallas import tpu_sc as plsc

B = 4096
S = 200
E = 64
C = 2
V = 1000000

VB = 32768
NW = 32
BPW = B // NW
NJ = BPW // 16


def _proj_body(w_ref, tt_ref, p_ref):
    w = w_ref[...] * (1.0 / S)
    chunk = tt_ref[...]
    res = lax.dot_general(w, chunk, (((0,), (0,)), ((), ())),
                          preferred_element_type=jnp.float32)
    u0 = lax.bitcast_convert_type(res[0].astype(jnp.bfloat16),
                                  jnp.uint16).astype(jnp.uint32)
    u1 = lax.bitcast_convert_type(res[1].astype(jnp.bfloat16),
                                  jnp.uint16).astype(jnp.uint32)
    p_ref[...] = (u0 | (u1 << 16)).astype(jnp.int32)


def _sc_pool(ids_hbm, mask_hbm, p_hbm, bias_hbm, out_hbm,
             idx_v, mask_v, r_v, bias_v, out_v, gsem):
    wid = lax.axis_index("s") * 2 + lax.axis_index("c")
    base = wid * BPW

    pltpu.sync_copy(ids_hbm.at[:, pl.ds(base, BPW)], idx_v)
    pltpu.sync_copy(mask_hbm.at[:, pl.ds(base, BPW)], mask_v)
    pltpu.sync_copy(bias_hbm, bias_v)

    def issue_body(s, _):
        pltpu.async_copy(p_hbm.at[idx_v.at[s, :]], r_v.at[s, :], gsem)
        return 0

    lax.fori_loop(0, S, issue_body, 0)

    zero = jnp.zeros((16,), jnp.float32)

    def drain_body(s, accs):
        pltpu.make_async_copy(p_hbm.at[idx_v.at[s, :]], r_v.at[s, :],
                              gsem).wait()
        out = []
        for j in range(NJ):
            m = mask_v[s, pl.ds(j * 16, 16)]
            pair = plsc.bitcast(r_v[s, pl.ds(j * 16, 16)], jnp.bfloat16)
            p0h, p1h = plsc.unpack(pair, format=plsc.PackFormat.INTERLEAVED)
            out.append(accs[2 * j] + m * p0h.astype(jnp.float32))
            out.append(accs[2 * j + 1] + m * p1h.astype(jnp.float32))
        return tuple(out)

    accs = lax.fori_loop(0, S, drain_body, (zero,) * (2 * NJ))
    b0 = bias_v[0, pl.ds(0, 16)]
    b1 = bias_v[1, pl.ds(0, 16)]
    for j in range(NJ):
        out_v[0, pl.ds(j * 16, 16)] = accs[2 * j] + b0
        out_v[1, pl.ds(j * 16, 16)] = accs[2 * j + 1] + b1
    pltpu.sync_copy(out_v, out_hbm.at[:, pl.ds(base, BPW)])


def kernel(input_ids, mask, emb_table, fc_w, fc_b):
    ids_t = input_ids.astype(jnp.int32).T
    mask_t = mask.T

    nblocks = pl.cdiv(V, VB)
    p_packed = pl.pallas_call(
        _proj_body,
        grid=(nblocks,),
        in_specs=[
            pl.BlockSpec((E, C), lambda i: (0, 0)),
            pl.BlockSpec((E, VB), lambda i: (0, i)),
        ],
        out_specs=pl.BlockSpec((VB,), lambda i: (i,)),
        out_shape=jax.ShapeDtypeStruct((V,), jnp.int32),
    )(fc_w, emb_table.T)

    mesh = plsc.VectorSubcoreMesh(core_axis_name="c", subcore_axis_name="s")
    pooled = pl.kernel(
        _sc_pool,
        out_type=jax.ShapeDtypeStruct((C, B), jnp.float32),
        mesh=mesh,
        compiler_params=pltpu.CompilerParams(needs_layout_passes=False,
                                             use_tc_tiling_on_sc=False),
        scratch_types=[
            pltpu.VMEM((S, BPW), jnp.int32),
            pltpu.VMEM((S, BPW), jnp.float32),
            pltpu.VMEM((S, BPW), jnp.int32),
            pltpu.VMEM((C, 16), jnp.float32),
            pltpu.VMEM((C, BPW), jnp.float32),
            pltpu.SemaphoreType.DMA,
        ],
    )(ids_t, mask_t, p_packed, jnp.broadcast_to(fc_b.reshape(C, 1), (C, 16)))

    return pooled.T


# speedup vs baseline: 1.0192x; 1.0004x over previous
import jax
import jax.numpy as jnp
from jax import lax
from jax.experimental import pallas as pl
from jax.experimental.pallas import tpu as pltpu
from jax.experimental.pallas import tpu_sc as plsc

B = 4096
S = 200
E = 64
C = 2
V = 1000000

VB = 32768
NW = 32
BPW = B // NW
NJ = BPW // 16


def _proj_body(w_ref, tt_ref, p_ref):
    w = w_ref[...] * (1.0 / S)
    chunk = tt_ref[...]
    res = lax.dot_general(w, chunk, (((0,), (0,)), ((), ())),
                          preferred_element_type=jnp.float32)
    u0 = lax.bitcast_convert_type(res[0].astype(jnp.bfloat16),
                                  jnp.uint16).astype(jnp.uint32)
    u1 = lax.bitcast_convert_type(res[1].astype(jnp.bfloat16),
                                  jnp.uint16).astype(jnp.uint32)
    p_ref[...] = (u0 | (u1 << 16)).astype(jnp.int32)


def _sc_pool(ids_hbm, mask_hbm, p_hbm, bias_hbm, out_hbm,
             idx_v, mask_v, r_v, bias_v, out_v, gsem):
    wid = lax.axis_index("s") * 2 + lax.axis_index("c")
    base = wid * BPW

    pltpu.sync_copy(ids_hbm.at[:, pl.ds(base, BPW)], idx_v)
    pltpu.sync_copy(mask_hbm.at[:, pl.ds(base, BPW)], mask_v)
    pltpu.sync_copy(bias_hbm, bias_v)

    def issue_body(s, _):
        pltpu.async_copy(p_hbm.at[idx_v.at[s, :]], r_v.at[s, :], gsem)
        return 0

    lax.fori_loop(0, S, issue_body, 0)

    zero = jnp.zeros((16,), jnp.float32)

    def drain_body(s, accs):
        pltpu.make_async_copy(p_hbm.at[idx_v.at[s, :]], r_v.at[s, :],
                              gsem).wait()
        out = []
        for j in range(NJ):
            m = mask_v[s, pl.ds(j * 16, 16)]
            pair = plsc.bitcast(r_v[s, pl.ds(j * 16, 16)], jnp.bfloat16)
            p0h, p1h = plsc.unpack(pair, format=plsc.PackFormat.INTERLEAVED)
            out.append(accs[2 * j] + m * p0h.astype(jnp.float32))
            out.append(accs[2 * j + 1] + m * p1h.astype(jnp.float32))
        return tuple(out)

    accs = lax.fori_loop(0, S, drain_body, (zero,) * (2 * NJ))
    b0 = bias_v[0, pl.ds(0, 16)]
    b1 = bias_v[1, pl.ds(0, 16)]
    for j in range(NJ):
        out_v[0, pl.ds(j * 16, 16)] = accs[2 * j] + b0
        out_v[1, pl.ds(j * 16, 16)] = accs[2 * j + 1] + b1
    pltpu.sync_copy(out_v, out_hbm.at[:, pl.ds(base, BPW)])


def kernel(input_ids, mask, emb_table, fc_w, fc_b):
    ids_t = input_ids.astype(jnp.int32).T
    mask_t = mask.T

    nblocks = pl.cdiv(V, VB)
    p_packed = pl.pallas_call(
        _proj_body,
        grid=(nblocks,),
        in_specs=[
            pl.BlockSpec((E, C), lambda i: (0, 0)),
            pl.BlockSpec((E, VB), lambda i: (0, i)),
        ],
        out_specs=pl.BlockSpec((VB,), lambda i: (i,)),
        out_shape=jax.ShapeDtypeStruct((V,), jnp.int32),
    )(fc_w, emb_table.T)

    mesh = plsc.VectorSubcoreMesh(core_axis_name="c", subcore_axis_name="s")
    pooled = pl.kernel(
        _sc_pool,
        out_type=jax.ShapeDtypeStruct((C, B), jnp.float32),
        mesh=mesh,
        compiler_params=pltpu.CompilerParams(needs_layout_passes=False,
                                             use_tc_tiling_on_sc=False),
        scratch_types=[
            pltpu.VMEM((S, BPW), jnp.int32),
            pltpu.VMEM((S, BPW), jnp.float32),
            pltpu.VMEM((S, BPW), jnp.int32),
            pltpu.VMEM((C, 16), jnp.float32),
            pltpu.VMEM((C, BPW), jnp.float32),
            pltpu.SemaphoreType.DMA,
        ],
    )(ids_t, mask_t, p_packed, jnp.broadcast_to(fc_b.reshape(C, 1), (C, 16)))

    return pooled.T


# speedup vs baseline: 1.0207x; 1.0015x over previous
import jax
import jax.numpy as jnp
from jax import lax
from jax.experimental import pallas as pl
from jax.experimental.pallas import tpu as pltpu
from jax.experimental.pallas import tpu_sc as plsc

B = 4096
S = 200
E = 64
C = 2
V = 1000000

VB = 32768
NW = 32
BPW = B // NW
NJ = BPW // 16


def _proj_body(w_ref, tt_ref, p_ref):
    w = w_ref[...] * (1.0 / S)
    chunk = tt_ref[...]
    res = lax.dot_general(w, chunk, (((0,), (0,)), ((), ())),
                          preferred_element_type=jnp.float32)
    u0 = lax.bitcast_convert_type(res[0].astype(jnp.bfloat16),
                                  jnp.uint16).astype(jnp.uint32)
    u1 = lax.bitcast_convert_type(res[1].astype(jnp.bfloat16),
                                  jnp.uint16).astype(jnp.uint32)
    p_ref[...] = (u0 | (u1 << 16)).astype(jnp.int32)


def _sc_pool(ids_hbm, mask_hbm, p_hbm, bias_hbm, out_hbm,
             idx_v, mask_v, r_v, bias_v, out_v, gsem):
    wid = lax.axis_index("s") * 2 + lax.axis_index("c")
    base = wid * BPW

    pltpu.sync_copy(ids_hbm.at[:, pl.ds(base, BPW)], idx_v)
    pltpu.sync_copy(mask_hbm.at[:, pl.ds(base, BPW)], mask_v)
    pltpu.sync_copy(bias_hbm, bias_v)

    def issue_body(s4, _):
        for k in range(4):
            s = s4 * 4 + k
            pltpu.async_copy(p_hbm.at[idx_v.at[s, :]], r_v.at[s, :], gsem)
        return 0

    lax.fori_loop(0, S // 4, issue_body, 0)

    zero = jnp.zeros((16,), jnp.float32)
    CD = 10

    def drain_body(r, accs):
        s0 = r * CD
        pltpu.make_async_copy(ids_hbm.at[pl.ds(0, CD), pl.ds(0, BPW)],
                              r_v.at[pl.ds(s0, CD), :], gsem).wait()
        for ss in range(CD):
            s = s0 + ss
            out = []
            for j in range(NJ):
                m = mask_v[s, pl.ds(j * 16, 16)]
                pair = plsc.bitcast(r_v[s, pl.ds(j * 16, 16)], jnp.bfloat16)
                p0h, p1h = plsc.unpack(pair, format=plsc.PackFormat.INTERLEAVED)
                out.append(accs[2 * j] + m * p0h.astype(jnp.float32))
                out.append(accs[2 * j + 1] + m * p1h.astype(jnp.float32))
            accs = tuple(out)
        return accs

    accs = lax.fori_loop(0, S // CD, drain_body, (zero,) * (2 * NJ))
    b0 = bias_v[0, pl.ds(0, 16)]
    b1 = bias_v[1, pl.ds(0, 16)]
    for j in range(NJ):
        out_v[0, pl.ds(j * 16, 16)] = accs[2 * j] + b0
        out_v[1, pl.ds(j * 16, 16)] = accs[2 * j + 1] + b1
    pltpu.sync_copy(out_v, out_hbm.at[:, pl.ds(base, BPW)])


def kernel(input_ids, mask, emb_table, fc_w, fc_b):
    ids_t = input_ids.astype(jnp.int32).T
    mask_t = mask.T

    nblocks = pl.cdiv(V, VB)
    p_packed = pl.pallas_call(
        _proj_body,
        grid=(nblocks,),
        in_specs=[
            pl.BlockSpec((E, C), lambda i: (0, 0)),
            pl.BlockSpec((E, VB), lambda i: (0, i)),
        ],
        out_specs=pl.BlockSpec((VB,), lambda i: (i,)),
        out_shape=jax.ShapeDtypeStruct((V,), jnp.int32),
    )(fc_w, emb_table.T)

    mesh = plsc.VectorSubcoreMesh(core_axis_name="c", subcore_axis_name="s")
    pooled = pl.kernel(
        _sc_pool,
        out_type=jax.ShapeDtypeStruct((C, B), jnp.float32),
        mesh=mesh,
        compiler_params=pltpu.CompilerParams(needs_layout_passes=False,
                                             use_tc_tiling_on_sc=False),
        scratch_types=[
            pltpu.VMEM((S, BPW), jnp.int32),
            pltpu.VMEM((S, BPW), jnp.float32),
            pltpu.VMEM((S, BPW), jnp.int32),
            pltpu.VMEM((C, 16), jnp.float32),
            pltpu.VMEM((C, BPW), jnp.float32),
            pltpu.SemaphoreType.DMA,
        ],
    )(ids_t, mask_t, p_packed, jnp.broadcast_to(fc_b.reshape(C, 1), (C, 16)))

    return pooled.T
